# baseline (speedup 1.0000x reference)
import functools

import jax
import jax.numpy as jnp
from jax import lax
from jax.experimental import pallas as pl
from jax.experimental.pallas import tpu as pltpu
from jax.experimental.pallas import tpu_sc as plsc

B = 16384
D = 32
W = 128
N = 1000000
KC_FULL = N // W
TAIL = N - KC_FULL * W
NC = 2
NS = 16
NW = NC * NS
BPW = B // NW
L = 16
KC_PER_W = 245


def _tr_body(src_hbm, out_hbm, buf0, buf1, buf2, buf3,
             tbuf0, tbuf1, tbuf2, tbuf3, tailbuf,
             sem0, sem1, sem2, sem3, osem0, osem1, osem2, osem3):
    wid = lax.axis_index("s") * NC + lax.axis_index("c")
    base_kc = wid * 244 + jnp.minimum(wid, 4)
    count = jnp.where(wid < 4, 245, 244)

    bufs = (buf0, buf1, buf2, buf3)
    tbufs = (tbuf0, tbuf1, tbuf2, tbuf3)
    sems = (sem0, sem1, sem2, sem3)
    osems = (osem0, osem1, osem2, osem3)

    def clamped_kc(t):
        return base_kc + jnp.minimum(t, count - 1)

    def in_copy(t, par):
        start = pl.multiple_of(clamped_kc(t) * W, W)
        return pltpu.make_async_copy(
            src_hbm.at[:, :, pl.ds(start, W)], bufs[par], sems[par])

    def out_copy(t, par):
        start = pl.multiple_of(clamped_kc(t) * (D * W), D * W)
        return pltpu.make_async_copy(
            tbufs[par], out_hbm.at[pl.ds(start, D * W)], osems[par])

    iota = lax.iota(jnp.int32, L)
    iota32 = iota * D

    def transpose(par, n_lg=W // L):
        buf = bufs[par]
        tbuf = tbufs[par]
        for d in range(D):
            dg, ds = d // 8, d % 8
            for lg in range(n_lg):
                v = buf[dg, ds, pl.ds(lg * L, L)]
                col = jnp.bitwise_and(iota + ((lg % 2) * L + d), D - 1)
                idx = (iota32 + lg * L * D) + col
                plsc.store_scatter(tbuf, [idx], v)

    in_copy(0, 0).start()
    in_copy(1, 1).start()
    in_copy(2, 2).start()

    def quad_body(g, _):
        for par in range(4):
            t = g * 4 + par

            @pl.when(t + 3 < KC_PER_W)
            def _():
                in_copy(t + 3, (par + 3) % 4).start()

            in_copy(t, par).wait()

            @pl.when(t >= 4)
            def _():
                out_copy(t - 4, par).wait()

            transpose(par)
            out_copy(t, par).start()
        return 0

    lax.fori_loop(0, 61, quad_body, 0)

    in_copy(244, 0).wait()
    out_copy(240, 0).wait()
    transpose(0)
    out_copy(244, 0).start()
    out_copy(241, 1).wait()
    out_copy(242, 2).wait()
    out_copy(243, 3).wait()
    out_copy(244, 0).wait()

    @pl.when(wid == NW - 1)
    def _():
        pltpu.make_async_copy(
            src_hbm.at[:, :, pl.ds(KC_FULL * W, TAIL)], tailbuf,
            sems[0]).start()
        pltpu.make_async_copy(
            src_hbm.at[:, :, pl.ds(KC_FULL * W, TAIL)], tailbuf,
            sems[0]).wait()
        for d in range(D):
            dg, ds = d // 8, d % 8
            for lg in range(TAIL // L):
                v = tailbuf[dg, ds, pl.ds(lg * L, L)]
                col = jnp.bitwise_and(iota + ((lg % 2) * L + d), D - 1)
                idx = (iota32 + lg * L * D) + col
                plsc.store_scatter(tbuf0, [idx], v)
        pltpu.make_async_copy(
            tbuf0.at[pl.ds(0, TAIL * D)],
            out_hbm.at[pl.ds(KC_FULL * W * D, TAIL * D)], osems[0]).start()
        pltpu.make_async_copy(
            tbuf0.at[pl.ds(0, TAIL * D)],
            out_hbm.at[pl.ds(KC_FULL * W * D, TAIL * D)], osems[0]).wait()


_tr_kernel = functools.partial(
    pl.kernel,
    out_type=jax.ShapeDtypeStruct((N * D,), jnp.float32),
    mesh=plsc.VectorSubcoreMesh(core_axis_name="c", subcore_axis_name="s"),
    scratch_types=[
        pltpu.VMEM((4, 8, W), jnp.float32),
        pltpu.VMEM((4, 8, W), jnp.float32),
        pltpu.VMEM((4, 8, W), jnp.float32),
        pltpu.VMEM((4, 8, W), jnp.float32),
        pltpu.VMEM((D * W,), jnp.float32),
        pltpu.VMEM((D * W,), jnp.float32),
        pltpu.VMEM((D * W,), jnp.float32),
        pltpu.VMEM((D * W,), jnp.float32),
        pltpu.VMEM((4, 8, TAIL), jnp.float32),
        pltpu.SemaphoreType.DMA,
        pltpu.SemaphoreType.DMA,
        pltpu.SemaphoreType.DMA,
        pltpu.SemaphoreType.DMA,
        pltpu.SemaphoreType.DMA,
        pltpu.SemaphoreType.DMA,
        pltpu.SemaphoreType.DMA,
        pltpu.SemaphoreType.DMA,
    ],
    compiler_params=pltpu.CompilerParams(
        needs_layout_passes=False, use_tc_tiling_on_sc=True),
)(_tr_body)


def _bpr_body(user_hbm, i_hbm, j_hbm, ut_hbm, it_hbm, out_hbm,
              idx_u, idx_i, idx_j, u_rows, i_rows, j_rows, out_v, sem):
    wid = lax.axis_index("s") * NC + lax.axis_index("c")
    base = wid * BPW

    pltpu.sync_copy(user_hbm.at[pl.ds(base, BPW)], idx_u)
    pltpu.sync_copy(i_hbm.at[pl.ds(base, BPW)], idx_i)
    pltpu.sync_copy(j_hbm.at[pl.ds(base, BPW)], idx_j)

    cu = pltpu.async_copy(ut_hbm.at[idx_u], u_rows, sem)
    ci = pltpu.async_copy(it_hbm.at[idx_i], i_rows, sem)
    cj = pltpu.async_copy(it_hbm.at[idx_j], j_rows, sem)
    cu.wait()
    ci.wait()
    cj.wait()

    def block(blk, _):
        sl = pl.ds(blk * L, L)
        rows = blk * L + lax.iota(jnp.int32, L)
        ru = jnp.bitwise_and(idx_u[sl], D - 1)
        ri = jnp.bitwise_and(idx_i[sl], D - 1)
        rj = jnp.bitwise_and(idx_j[sl], D - 1)
        accs = [jnp.zeros((L,), jnp.float32) for _ in range(4)]
        for dd in range(D):
            du = jnp.bitwise_and(ru + dd, D - 1)
            di = jnp.bitwise_and(ri + dd, D - 1)
            dj = jnp.bitwise_and(rj + dd, D - 1)
            u_v = plsc.load_gather(u_rows, [rows, du])
            i_v = plsc.load_gather(i_rows, [rows, di])
            j_v = plsc.load_gather(j_rows, [rows, dj])
            accs[dd % 4] = accs[dd % 4] + u_v * (i_v - j_v)
        acc = (accs[0] + accs[1]) + (accs[2] + accs[3])
        out_v[pl.ds(blk * L, L)] = acc
        return 0

    lax.fori_loop(0, BPW // L, block, 0)

    pltpu.sync_copy(out_v, out_hbm.at[pl.ds(base, BPW)])


_bpr_kernel = functools.partial(
    pl.kernel,
    out_type=jax.ShapeDtypeStruct((B,), jnp.float32),
    mesh=plsc.VectorSubcoreMesh(core_axis_name="c", subcore_axis_name="s"),
    scratch_types=[
        pltpu.VMEM((BPW,), jnp.int32),
        pltpu.VMEM((BPW,), jnp.int32),
        pltpu.VMEM((BPW,), jnp.int32),
        pltpu.VMEM((BPW, D), jnp.float32),
        pltpu.VMEM((BPW, D), jnp.float32),
        pltpu.VMEM((BPW, D), jnp.float32),
        pltpu.VMEM((BPW,), jnp.float32),
        pltpu.SemaphoreType.DMA,
    ],
    compiler_params=pltpu.CompilerParams(
        needs_layout_passes=False, use_tc_tiling_on_sc=False),
)(_bpr_body)


def kernel(user, i, j, user_table, item_table):
    n = user_table.shape[0]
    ut3 = user_table.T.reshape(4, 8, n)
    it3 = item_table.T.reshape(4, 8, n)
    ut_r = _tr_kernel(ut3).reshape(n, D)
    it_r = _tr_kernel(it3).reshape(n, D)
    return _bpr_kernel(user, i, j, ut_r, it_r)


# speedup vs baseline: 1.2410x; 1.2410x over previous
import functools

import jax
import jax.numpy as jnp
from jax import lax
from jax.experimental import pallas as pl
from jax.experimental.pallas import tpu as pltpu
from jax.experimental.pallas import tpu_sc as plsc

B = 16384
D = 32
W = 128
N = 1000000
KC_FULL = N // W
TAIL = N - KC_FULL * W
NC = 2
NS = 16
NW = NC * NS
BPW = B // NW
L = 16
KC_PER_W = 245


def _tr_body(src_hbm, out_hbm, buf0, buf1, tbuf0, tbuf1, tailbuf,
             sem0, sem1, osem0, osem1):
    wid = lax.axis_index("s") * NC + lax.axis_index("c")
    base_kc = wid * 244 + jnp.minimum(wid, 4)
    count = jnp.where(wid < 4, 245, 244)

    bufs = (buf0, buf1)
    tbufs = (tbuf0, tbuf1)
    sems = (sem0, sem1)
    osems = (osem0, osem1)

    def clamped_kc(t):
        return base_kc + jnp.minimum(t, count - 1)

    def in_copy(t, par):
        start = pl.multiple_of(clamped_kc(t) * W, W)
        return pltpu.make_async_copy(
            src_hbm.at[:, :, pl.ds(start, W)], bufs[par], sems[par])

    def out_copy(t, par):
        start = pl.multiple_of(clamped_kc(t) * (D * W), D * W)
        return pltpu.make_async_copy(
            tbufs[par], out_hbm.at[pl.ds(start, D * W)], osems[par])

    iota = lax.iota(jnp.int32, L)
    iota32 = iota * D

    def transpose(par, n_lg=W // L):
        buf = bufs[par]
        tbuf = tbufs[par]
        for d in range(D):
            dg, ds = d // 8, d % 8
            for lg in range(n_lg):
                v = buf[dg, ds, pl.ds(lg * L, L)]
                col = jnp.bitwise_and(iota + ((lg % 2) * L + d), D - 1)
                idx = (iota32 + lg * L * D) + col
                plsc.store_scatter(tbuf, [idx], v)

    in_copy(0, 0).start()

    def pair_body(g, _):
        for par in range(2):
            t = g * 2 + par
            in_copy(t + 1, (par + 1) % 2).start()
            in_copy(t, par).wait()

            @pl.when(t >= 2)
            def _():
                out_copy(t - 2, par).wait()

            transpose(par)
            out_copy(t, par).start()
        return 0

    lax.fori_loop(0, 122, pair_body, 0)

    in_copy(244, 0).wait()
    out_copy(242, 0).wait()
    transpose(0)
    out_copy(244, 0).start()
    out_copy(243, 1).wait()
    out_copy(244, 0).wait()

    @pl.when(wid == NW - 1)
    def _():
        pltpu.make_async_copy(
            src_hbm.at[:, :, pl.ds(KC_FULL * W, TAIL)], tailbuf,
            sems[0]).start()
        pltpu.make_async_copy(
            src_hbm.at[:, :, pl.ds(KC_FULL * W, TAIL)], tailbuf,
            sems[0]).wait()
        for d in range(D):
            dg, ds = d // 8, d % 8
            for lg in range(TAIL // L):
                v = tailbuf[dg, ds, pl.ds(lg * L, L)]
                col = jnp.bitwise_and(iota + ((lg % 2) * L + d), D - 1)
                idx = (iota32 + lg * L * D) + col
                plsc.store_scatter(tbuf0, [idx], v)
        pltpu.make_async_copy(
            tbuf0.at[pl.ds(0, TAIL * D)],
            out_hbm.at[pl.ds(KC_FULL * W * D, TAIL * D)], osems[0]).start()
        pltpu.make_async_copy(
            tbuf0.at[pl.ds(0, TAIL * D)],
            out_hbm.at[pl.ds(KC_FULL * W * D, TAIL * D)], osems[0]).wait()


_tr_kernel = functools.partial(
    pl.kernel,
    out_type=jax.ShapeDtypeStruct((N * D,), jnp.float32),
    mesh=plsc.VectorSubcoreMesh(core_axis_name="c", subcore_axis_name="s"),
    scratch_types=[
        pltpu.VMEM((4, 8, W), jnp.float32),
        pltpu.VMEM((4, 8, W), jnp.float32),
        pltpu.VMEM((D * W,), jnp.float32),
        pltpu.VMEM((D * W,), jnp.float32),
        pltpu.VMEM((4, 8, TAIL), jnp.float32),
        pltpu.SemaphoreType.DMA,
        pltpu.SemaphoreType.DMA,
        pltpu.SemaphoreType.DMA,
        pltpu.SemaphoreType.DMA,
    ],
    compiler_params=pltpu.CompilerParams(
        needs_layout_passes=False, use_tc_tiling_on_sc=True),
)(_tr_body)


def _bpr_body(user_hbm, i_hbm, j_hbm, ut_hbm, it_hbm, out_hbm,
              idx_u, idx_i, idx_j, u_rows, i_rows, j_rows, out_v, sem):
    wid = lax.axis_index("s") * NC + lax.axis_index("c")
    base = wid * BPW

    pltpu.sync_copy(user_hbm.at[pl.ds(base, BPW)], idx_u)
    pltpu.sync_copy(i_hbm.at[pl.ds(base, BPW)], idx_i)
    pltpu.sync_copy(j_hbm.at[pl.ds(base, BPW)], idx_j)

    cu = pltpu.async_copy(ut_hbm.at[idx_u], u_rows, sem)
    ci = pltpu.async_copy(it_hbm.at[idx_i], i_rows, sem)
    cj = pltpu.async_copy(it_hbm.at[idx_j], j_rows, sem)
    cu.wait()
    ci.wait()
    cj.wait()

    def block(blk, _):
        sl = pl.ds(blk * L, L)
        rows = blk * L + lax.iota(jnp.int32, L)
        ru = jnp.bitwise_and(idx_u[sl], D - 1)
        ri = jnp.bitwise_and(idx_i[sl], D - 1)
        rj = jnp.bitwise_and(idx_j[sl], D - 1)
        accs = [jnp.zeros((L,), jnp.float32) for _ in range(4)]
        for dd in range(D):
            du = jnp.bitwise_and(ru + dd, D - 1)
            di = jnp.bitwise_and(ri + dd, D - 1)
            dj = jnp.bitwise_and(rj + dd, D - 1)
            u_v = plsc.load_gather(u_rows, [rows, du])
            i_v = plsc.load_gather(i_rows, [rows, di])
            j_v = plsc.load_gather(j_rows, [rows, dj])
            accs[dd % 4] = accs[dd % 4] + u_v * (i_v - j_v)
        acc = (accs[0] + accs[1]) + (accs[2] + accs[3])
        out_v[pl.ds(blk * L, L)] = acc
        return 0

    lax.fori_loop(0, BPW // L, block, 0)

    pltpu.sync_copy(out_v, out_hbm.at[pl.ds(base, BPW)])


_bpr_kernel = functools.partial(
    pl.kernel,
    out_type=jax.ShapeDtypeStruct((B,), jnp.float32),
    mesh=plsc.VectorSubcoreMesh(core_axis_name="c", subcore_axis_name="s"),
    scratch_types=[
        pltpu.VMEM((BPW,), jnp.int32),
        pltpu.VMEM((BPW,), jnp.int32),
        pltpu.VMEM((BPW,), jnp.int32),
        pltpu.VMEM((BPW, D), jnp.float32),
        pltpu.VMEM((BPW, D), jnp.float32),
        pltpu.VMEM((BPW, D), jnp.float32),
        pltpu.VMEM((BPW,), jnp.float32),
        pltpu.SemaphoreType.DMA,
    ],
    compiler_params=pltpu.CompilerParams(
        needs_layout_passes=False, use_tc_tiling_on_sc=False),
)(_bpr_body)


def kernel(user, i, j, user_table, item_table):
    n = user_table.shape[0]
    ut3 = user_table.T.reshape(4, 8, n)
    it3 = item_table.T.reshape(4, 8, n)
    ut_r = _tr_kernel(ut3).reshape(n, D)
    it_r = _tr_kernel(it3).reshape(n, D)
    return _bpr_kernel(user, i, j, ut_r, it_r)
